# baseline (speedup 1.0000x reference)
import functools

import jax
import jax.numpy as jnp
from jax import lax
from jax.experimental import pallas as pl
from jax.experimental.pallas import tpu as pltpu
from jax.experimental.pallas import tpu_sc as plsc

T = 4
N = 10000
E = 320000
B = 16
F = 128
NC = 16

NCORE = 2
NSUB = 16
NTILE = NCORE * NSUB
EPT = E // NTILE
K = 80
NCHUNK = EPT // K
RCH = 200
NRCH = N // RCH


def _agg_body(with_cnt, *refs):
    if with_cnt:
        (table, esrc, edst, zf, zc, onesh, out, cnt_out,
         acc, cacc, srca, dstv0, dstv1, gbuf0, gbuf1, onesb,
         gsem0, gsem1, ssem0, ssem1, csem0, csem1, isem0, isem1) = refs
    else:
        (table, esrc, edst, zf, out,
         acc, srca, dstv0, dstv1, gbuf0, gbuf1,
         gsem0, gsem1, ssem0, ssem1, isem0, isem1) = refs
    c = lax.axis_index("c")
    s = lax.axis_index("s")
    wid = c * NSUB + s
    nfull = NRCH // NSUB
    nextra = NRCH - nfull * NSUB

    if with_cnt:
        pltpu.sync_copy(onesh, onesb)

    def g_desc(ch, buf, sem):
        off = pl.multiple_of(ch * K, 8)
        return pltpu.make_async_copy(
            table.at[srca.at[pl.ds(off, K)]], buf, sem)

    def i_desc(ch, buf, sem):
        return pltpu.make_async_copy(edst.at[wid, ch], buf, sem)

    pltpu.sync_copy(esrc.at[pl.ds(wid * EPT, EPT)], srca)

    def _zero(rr):
        pltpu.sync_copy(zf, acc.at[pl.ds(rr, RCH)])
        if with_cnt:
            pltpu.sync_copy(zc, cacc.at[pl.ds(rr, RCH)])

    for j in range(nfull):
        _zero(pl.multiple_of((s + j * NSUB) * RCH, 8))

    @pl.when(s < nextra)
    def _zx():
        _zero(pl.multiple_of((s + nfull * NSUB) * RCH, 8))
    plsc.subcore_barrier()

    pltpu.sync_copy(edst.at[wid, 0], dstv0)
    g_desc(0, gbuf0, gsem0).start()
    g_desc(1, gbuf1, gsem1).start()
    i_desc(1, dstv1, isem1).start()

    def body2(it, carry):
        g = it * 2
        g_desc(g, gbuf0, gsem0).wait()
        d0 = pltpu.async_copy(gbuf0, acc.at[dstv0], ssem0, add=True)
        if with_cnt:
            c0 = pltpu.async_copy(onesb, cacc.at[dstv0], csem0, add=True)
        i_desc(g + 1, dstv1, isem1).wait()
        g_desc(g + 1, gbuf1, gsem1).wait()
        d1 = pltpu.async_copy(gbuf1, acc.at[dstv1], ssem1, add=True)
        if with_cnt:
            c1 = pltpu.async_copy(onesb, cacc.at[dstv1], csem1, add=True)
        d0.wait()
        if with_cnt:
            c0.wait()

        @pl.when(g + 2 < NCHUNK)
        def _n0():
            i_desc(g + 2, dstv0, isem0).start()
            g_desc(g + 2, gbuf0, gsem0).start()
        d1.wait()
        if with_cnt:
            c1.wait()

        @pl.when(g + 2 < NCHUNK)
        def _w0():
            i_desc(g + 2, dstv0, isem0).wait()

        @pl.when(g + 3 < NCHUNK)
        def _n1():
            i_desc(g + 3, dstv1, isem1).start()
            g_desc(g + 3, gbuf1, gsem1).start()
        return carry

    lax.fori_loop(0, NCHUNK // 2, body2, 0)
    g_desc(NCHUNK - 1, gbuf0, gsem0).wait()
    dl = pltpu.async_copy(gbuf0, acc.at[dstv0], ssem0, add=True)
    if with_cnt:
        cl = pltpu.async_copy(onesb, cacc.at[dstv0], csem0, add=True)
    dl.wait()
    if with_cnt:
        cl.wait()
    plsc.subcore_barrier()

    def _wout(rr):
        pltpu.sync_copy(acc.at[pl.ds(rr, RCH)], out.at[c, pl.ds(rr, RCH)])
        if with_cnt:
            pltpu.sync_copy(cacc.at[pl.ds(rr, RCH)],
                            cnt_out.at[c, pl.ds(rr, RCH)])

    for j in range(nfull):
        _wout(pl.multiple_of((s + j * NSUB) * RCH, 8))

    @pl.when(s < nextra)
    def _wx():
        _wout(pl.multiple_of((s + nfull * NSUB) * RCH, 8))


def _make_agg(with_cnt):
    mesh = plsc.VectorSubcoreMesh(core_axis_name="c", subcore_axis_name="s")
    out_type = [jax.ShapeDtypeStruct((NCORE, N, F), jnp.float32)]
    scratch = [
        pltpu.VMEM_SHARED((N, F), jnp.float32),
    ]
    if with_cnt:
        out_type.append(jax.ShapeDtypeStruct((NCORE, N, 16), jnp.float32))
        scratch.append(pltpu.VMEM_SHARED((N, 16), jnp.float32))
    scratch += [
        pltpu.VMEM((EPT,), jnp.int32),
        pltpu.VMEM((K,), jnp.int32),
        pltpu.VMEM((K,), jnp.int32),
        pltpu.VMEM((K, F), jnp.float32),
        pltpu.VMEM((K, F), jnp.float32),
    ]
    if with_cnt:
        scratch.append(pltpu.VMEM((K, 16), jnp.float32))
    nsem = 8 if with_cnt else 6
    scratch += [pltpu.SemaphoreType.DMA] * nsem
    return pl.kernel(
        functools.partial(_agg_body, with_cnt),
        out_type=tuple(out_type) if with_cnt else out_type[0],
        mesh=mesh,
        scratch_types=scratch,
        compiler_params=pltpu.CompilerParams(use_tc_tiling_on_sc=False),
    )


BLKR = 400
NBLK = N // BLKR


def _sage1_body(s_ref, c_ref, x_ref, wl_ref, b_ref, wr_ref, o_ref):
    ssum = s_ref[0] + s_ref[1]
    cnt = c_ref[0, :, 0:1] + c_ref[1, :, 0:1]
    m = ssum * (1.0 / jnp.maximum(cnt, 1.0))
    h = (jnp.dot(m, wl_ref[...], preferred_element_type=jnp.float32)
         + b_ref[...]
         + jnp.dot(x_ref[...], wr_ref[...], preferred_element_type=jnp.float32))
    o_ref[...] = jnp.maximum(h, 0.0)


def _sage2_body(s_ref, c_ref, x_ref, wl_ref, b_ref, wr_ref, bat_ref, o_ref):
    r = pl.program_id(0)

    @pl.when(r == 0)
    def _init():
        o_ref[...] = jnp.full((B, F), -jnp.inf, jnp.float32)

    ssum = s_ref[0] + s_ref[1]
    cnt = c_ref[0, :, 0:1] + c_ref[1, :, 0:1]
    m = ssum * (1.0 / jnp.maximum(cnt, 1.0))
    h = (jnp.dot(m, wl_ref[...], preferred_element_type=jnp.float32)
         + b_ref[...]
         + jnp.dot(x_ref[...], wr_ref[...], preferred_element_type=jnp.float32))
    bat = bat_ref[...]
    bmin = jnp.min(bat)
    bmax = jnp.max(bat)
    for b in range(B):
        @pl.when((b >= bmin) & (b <= bmax))
        def _upd(b=b):
            vals = jnp.where(bat == b, h, -jnp.inf)
            mb = jnp.max(vals, axis=0, keepdims=True)
            o_ref[b:b + 1, :] = jnp.maximum(o_ref[b:b + 1, :], mb)


def _gru_body(seq_ref, wir, wiz, win, whr, whz, whn, br, bz, bin_, bhn,
              o_ref):
    f32 = jnp.float32
    h = jnp.zeros((B, NC), f32)
    for t in range(T):
        xt = seq_ref[t]
        ir = jnp.dot(xt, wir[...], preferred_element_type=f32)
        iz = jnp.dot(xt, wiz[...], preferred_element_type=f32)
        in_ = jnp.dot(xt, win[...], preferred_element_type=f32)
        r = jax.nn.sigmoid(ir + jnp.dot(h, whr[...], preferred_element_type=f32) + br[...])
        z = jax.nn.sigmoid(iz + jnp.dot(h, whz[...], preferred_element_type=f32) + bz[...])
        n = jnp.tanh(in_ + bin_[...] + r * (jnp.dot(h, whn[...], preferred_element_type=f32) + bhn[...]))
        h = (1.0 - z) * n + z * h
        mx = jnp.max(h, axis=1, keepdims=True)
        lse = mx + jnp.log(jnp.sum(jnp.exp(h - mx), axis=1, keepdims=True))
        o_ref[t] = h - lse


def kernel(x, edge_index, batch, W1_l, b1_l, W1_r, W2_l, b2_l, W2_r,
           W_ih, W_hh, b_ih, b_hh):
    zf = jnp.zeros((RCH, F), jnp.float32)
    zc = jnp.zeros((RCH, 16), jnp.float32)
    onesh = jnp.ones((K, 16), jnp.float32)

    esrc = edge_index[:, 0, :]
    edst = edge_index[:, 1, :].reshape(T, NTILE, NCHUNK, K)

    agg1 = _make_agg(True)
    agg2 = _make_agg(False)

    w1l = W1_l.T
    w1r = W1_r.T
    b1 = b1_l.reshape(1, F)
    w2l = W2_l.T
    w2r = W2_r.T
    b2 = b2_l.reshape(1, F)
    bat2 = batch.reshape(N, 1)

    sage1 = pl.pallas_call(
        _sage1_body,
        grid=(NBLK,),
        in_specs=[
            pl.BlockSpec((NCORE, BLKR, F), lambda r: (0, r, 0)),
            pl.BlockSpec((NCORE, BLKR, 16), lambda r: (0, r, 0)),
            pl.BlockSpec((BLKR, F), lambda r: (r, 0)),
            pl.BlockSpec((F, F), lambda r: (0, 0)),
            pl.BlockSpec((1, F), lambda r: (0, 0)),
            pl.BlockSpec((F, F), lambda r: (0, 0)),
        ],
        out_specs=pl.BlockSpec((BLKR, F), lambda r: (r, 0)),
        out_shape=jax.ShapeDtypeStruct((N, F), jnp.float32),
    )
    sage2 = pl.pallas_call(
        _sage2_body,
        grid=(NBLK,),
        in_specs=[
            pl.BlockSpec((NCORE, BLKR, F), lambda r: (0, r, 0)),
            pl.BlockSpec((NCORE, BLKR, 16), lambda r: (0, r, 0)),
            pl.BlockSpec((BLKR, F), lambda r: (r, 0)),
            pl.BlockSpec((F, F), lambda r: (0, 0)),
            pl.BlockSpec((1, F), lambda r: (0, 0)),
            pl.BlockSpec((F, F), lambda r: (0, 0)),
            pl.BlockSpec((BLKR, 1), lambda r: (r, 0)),
        ],
        out_specs=pl.BlockSpec((B, F), lambda r: (0, 0)),
        out_shape=jax.ShapeDtypeStruct((B, F), jnp.float32),
    )

    seq_ts = []
    for t in range(T):
        s1, cnt = agg1(x[t], esrc[t], edst[t], zf, zc, onesh)
        h1 = sage1(s1, cnt, x[t], w1l, b1, w1r)
        s2 = agg2(h1, esrc[t], edst[t], zf)
        seq_ts.append(sage2(s2, cnt, h1, w2l, b2, w2r, bat2))
    seq = jnp.stack(seq_ts)

    wih = W_ih.T
    whh = W_hh.T
    wir, wiz, win = wih[:, 0:NC], wih[:, NC:2 * NC], wih[:, 2 * NC:3 * NC]
    whr, whz, whn = whh[:, 0:NC], whh[:, NC:2 * NC], whh[:, 2 * NC:3 * NC]
    br = (b_ih[0:NC] + b_hh[0:NC]).reshape(1, NC)
    bz = (b_ih[NC:2 * NC] + b_hh[NC:2 * NC]).reshape(1, NC)
    bin_ = b_ih[2 * NC:3 * NC].reshape(1, NC)
    bhn = b_hh[2 * NC:3 * NC].reshape(1, NC)

    out = pl.pallas_call(
        _gru_body,
        out_shape=jax.ShapeDtypeStruct((T, B, NC), jnp.float32),
    )(seq, wir, wiz, win, whr, whz, whn, br, bz, bin_, bhn)
    return out


# speedup vs baseline: 1.0437x; 1.0437x over previous
import functools

import jax
import jax.numpy as jnp
from jax import lax
from jax.experimental import pallas as pl
from jax.experimental.pallas import tpu as pltpu
from jax.experimental.pallas import tpu_sc as plsc

T = 4
N = 10000
E = 320000
B = 16
F = 128
NC = 16

NCORE = 2
NSUB = 16
NTILE = NCORE * NSUB
EPT = E // NTILE
K = 80
NCHUNK = EPT // K
RCH = 200
NRCH = N // RCH


def _agg_body(with_cnt, *refs):
    if with_cnt:
        (table, esrc, edst, zf, zc, onesh, out, cnt_out,
         acc, cacc, srca, dstv0, dstv1, gbuf0, gbuf1, onesb,
         gsem0, gsem1, ssem0, ssem1, csem0, csem1, isem0, isem1) = refs
    else:
        (table, esrc, edst, zf, out,
         acc, srca, dstv0, dstv1, gbuf0, gbuf1,
         gsem0, gsem1, ssem0, ssem1, isem0, isem1) = refs
    c = lax.axis_index("c")
    s = lax.axis_index("s")
    wid = c * NSUB + s
    nfull = NRCH // NSUB
    nextra = NRCH - nfull * NSUB

    if with_cnt:
        pltpu.sync_copy(onesh, onesb)

    def g_desc(ch, buf, sem):
        off = pl.multiple_of(ch * K, 8)
        return pltpu.make_async_copy(
            table.at[srca.at[pl.ds(off, K)]], buf, sem)

    def i_desc(ch, buf, sem):
        return pltpu.make_async_copy(edst.at[wid, ch], buf, sem)

    pltpu.sync_copy(esrc.at[pl.ds(wid * EPT, EPT)], srca)
    pltpu.sync_copy(edst.at[wid, 0], dstv0)
    g_desc(0, gbuf0, gsem0).start()
    i_desc(1, dstv1, isem1).start()

    def _zero(rr):
        pltpu.sync_copy(zf, acc.at[pl.ds(rr, RCH)])
        if with_cnt:
            pltpu.sync_copy(zc, cacc.at[pl.ds(rr, RCH)])

    for j in range(nfull):
        _zero(pl.multiple_of((s + j * NSUB) * RCH, 8))

    @pl.when(s < nextra)
    def _zx():
        _zero(pl.multiple_of((s + nfull * NSUB) * RCH, 8))
    plsc.subcore_barrier()


    def body2(it, carry):
        g = it * 2
        g_desc(g + 1, gbuf1, gsem1).start()
        g_desc(g, gbuf0, gsem0).wait()
        d0 = pltpu.async_copy(gbuf0, acc.at[dstv0], ssem0, add=True)
        if with_cnt:
            c0 = pltpu.async_copy(onesb, cacc.at[dstv0], csem0, add=True)
        i_desc(g + 1, dstv1, isem1).wait()
        d0.wait()
        if with_cnt:
            c0.wait()

        @pl.when(g + 2 < NCHUNK)
        def _n0():
            i_desc(g + 2, dstv0, isem0).start()
            g_desc(g + 2, gbuf0, gsem0).start()
        g_desc(g + 1, gbuf1, gsem1).wait()
        d1 = pltpu.async_copy(gbuf1, acc.at[dstv1], ssem1, add=True)
        if with_cnt:
            c1 = pltpu.async_copy(onesb, cacc.at[dstv1], csem1, add=True)

        @pl.when(g + 2 < NCHUNK)
        def _w0():
            i_desc(g + 2, dstv0, isem0).wait()
        d1.wait()
        if with_cnt:
            c1.wait()

        @pl.when(g + 3 < NCHUNK)
        def _n1():
            i_desc(g + 3, dstv1, isem1).start()
        return carry

    lax.fori_loop(0, NCHUNK // 2, body2, 0)
    g_desc(NCHUNK - 1, gbuf0, gsem0).wait()
    dl = pltpu.async_copy(gbuf0, acc.at[dstv0], ssem0, add=True)
    if with_cnt:
        cl = pltpu.async_copy(onesb, cacc.at[dstv0], csem0, add=True)
    dl.wait()
    if with_cnt:
        cl.wait()
    plsc.subcore_barrier()

    def _wout(rr):
        pltpu.sync_copy(acc.at[pl.ds(rr, RCH)], out.at[c, pl.ds(rr, RCH)])
        if with_cnt:
            pltpu.sync_copy(cacc.at[pl.ds(rr, RCH)],
                            cnt_out.at[c, pl.ds(rr, RCH)])

    for j in range(nfull):
        _wout(pl.multiple_of((s + j * NSUB) * RCH, 8))

    @pl.when(s < nextra)
    def _wx():
        _wout(pl.multiple_of((s + nfull * NSUB) * RCH, 8))


def _make_agg(with_cnt):
    mesh = plsc.VectorSubcoreMesh(core_axis_name="c", subcore_axis_name="s")
    out_type = [jax.ShapeDtypeStruct((NCORE, N, F), jnp.float32)]
    scratch = [
        pltpu.VMEM_SHARED((N, F), jnp.float32),
    ]
    if with_cnt:
        out_type.append(jax.ShapeDtypeStruct((NCORE, N, 16), jnp.float32))
        scratch.append(pltpu.VMEM_SHARED((N, 16), jnp.float32))
    scratch += [
        pltpu.VMEM((EPT,), jnp.int32),
        pltpu.VMEM((K,), jnp.int32),
        pltpu.VMEM((K,), jnp.int32),
        pltpu.VMEM((K, F), jnp.float32),
        pltpu.VMEM((K, F), jnp.float32),
    ]
    if with_cnt:
        scratch.append(pltpu.VMEM((K, 16), jnp.float32))
    nsem = 8 if with_cnt else 6
    scratch += [pltpu.SemaphoreType.DMA] * nsem
    return pl.kernel(
        functools.partial(_agg_body, with_cnt),
        out_type=tuple(out_type) if with_cnt else out_type[0],
        mesh=mesh,
        scratch_types=scratch,
        compiler_params=pltpu.CompilerParams(use_tc_tiling_on_sc=False),
    )


BLKR = 400
NBLK = N // BLKR


def _sage1_body(s_ref, c_ref, x_ref, wl_ref, b_ref, wr_ref, o_ref):
    ssum = s_ref[0] + s_ref[1]
    cnt = c_ref[0, :, 0:1] + c_ref[1, :, 0:1]
    m = ssum * (1.0 / jnp.maximum(cnt, 1.0))
    h = (jnp.dot(m, wl_ref[...], preferred_element_type=jnp.float32)
         + b_ref[...]
         + jnp.dot(x_ref[...], wr_ref[...], preferred_element_type=jnp.float32))
    o_ref[...] = jnp.maximum(h, 0.0)


def _sage2_body(s_ref, c_ref, x_ref, wl_ref, b_ref, wr_ref, bat_ref, o_ref):
    r = pl.program_id(0)

    @pl.when(r == 0)
    def _init():
        o_ref[...] = jnp.full((B, F), -jnp.inf, jnp.float32)

    ssum = s_ref[0] + s_ref[1]
    cnt = c_ref[0, :, 0:1] + c_ref[1, :, 0:1]
    m = ssum * (1.0 / jnp.maximum(cnt, 1.0))
    h = (jnp.dot(m, wl_ref[...], preferred_element_type=jnp.float32)
         + b_ref[...]
         + jnp.dot(x_ref[...], wr_ref[...], preferred_element_type=jnp.float32))
    bat = bat_ref[...]
    bmin = jnp.min(bat)
    bmax = jnp.max(bat)
    for b in range(B):
        @pl.when((b >= bmin) & (b <= bmax))
        def _upd(b=b):
            vals = jnp.where(bat == b, h, -jnp.inf)
            mb = jnp.max(vals, axis=0, keepdims=True)
            o_ref[b:b + 1, :] = jnp.maximum(o_ref[b:b + 1, :], mb)


def _gru_body(seq_ref, wir, wiz, win, whr, whz, whn, br, bz, bin_, bhn,
              o_ref):
    f32 = jnp.float32
    h = jnp.zeros((B, NC), f32)
    for t in range(T):
        xt = seq_ref[t]
        ir = jnp.dot(xt, wir[...], preferred_element_type=f32)
        iz = jnp.dot(xt, wiz[...], preferred_element_type=f32)
        in_ = jnp.dot(xt, win[...], preferred_element_type=f32)
        r = jax.nn.sigmoid(ir + jnp.dot(h, whr[...], preferred_element_type=f32) + br[...])
        z = jax.nn.sigmoid(iz + jnp.dot(h, whz[...], preferred_element_type=f32) + bz[...])
        n = jnp.tanh(in_ + bin_[...] + r * (jnp.dot(h, whn[...], preferred_element_type=f32) + bhn[...]))
        h = (1.0 - z) * n + z * h
        mx = jnp.max(h, axis=1, keepdims=True)
        lse = mx + jnp.log(jnp.sum(jnp.exp(h - mx), axis=1, keepdims=True))
        o_ref[t] = h - lse


def kernel(x, edge_index, batch, W1_l, b1_l, W1_r, W2_l, b2_l, W2_r,
           W_ih, W_hh, b_ih, b_hh):
    zf = jnp.zeros((RCH, F), jnp.float32)
    zc = jnp.zeros((RCH, 16), jnp.float32)
    onesh = jnp.ones((K, 16), jnp.float32)

    esrc = edge_index[:, 0, :]
    edst = edge_index[:, 1, :].reshape(T, NTILE, NCHUNK, K)

    agg1 = _make_agg(True)
    agg2 = _make_agg(False)

    w1l = W1_l.T
    w1r = W1_r.T
    b1 = b1_l.reshape(1, F)
    w2l = W2_l.T
    w2r = W2_r.T
    b2 = b2_l.reshape(1, F)
    bat2 = batch.reshape(N, 1)

    sage1 = pl.pallas_call(
        _sage1_body,
        grid=(NBLK,),
        in_specs=[
            pl.BlockSpec((NCORE, BLKR, F), lambda r: (0, r, 0)),
            pl.BlockSpec((NCORE, BLKR, 16), lambda r: (0, r, 0)),
            pl.BlockSpec((BLKR, F), lambda r: (r, 0)),
            pl.BlockSpec((F, F), lambda r: (0, 0)),
            pl.BlockSpec((1, F), lambda r: (0, 0)),
            pl.BlockSpec((F, F), lambda r: (0, 0)),
        ],
        out_specs=pl.BlockSpec((BLKR, F), lambda r: (r, 0)),
        out_shape=jax.ShapeDtypeStruct((N, F), jnp.float32),
    )
    sage2 = pl.pallas_call(
        _sage2_body,
        grid=(NBLK,),
        in_specs=[
            pl.BlockSpec((NCORE, BLKR, F), lambda r: (0, r, 0)),
            pl.BlockSpec((NCORE, BLKR, 16), lambda r: (0, r, 0)),
            pl.BlockSpec((BLKR, F), lambda r: (r, 0)),
            pl.BlockSpec((F, F), lambda r: (0, 0)),
            pl.BlockSpec((1, F), lambda r: (0, 0)),
            pl.BlockSpec((F, F), lambda r: (0, 0)),
            pl.BlockSpec((BLKR, 1), lambda r: (r, 0)),
        ],
        out_specs=pl.BlockSpec((B, F), lambda r: (0, 0)),
        out_shape=jax.ShapeDtypeStruct((B, F), jnp.float32),
    )

    seq_ts = []
    for t in range(T):
        s1, cnt = agg1(x[t], esrc[t], edst[t], zf, zc, onesh)
        h1 = sage1(s1, cnt, x[t], w1l, b1, w1r)
        s2 = agg2(h1, esrc[t], edst[t], zf)
        seq_ts.append(sage2(s2, cnt, h1, w2l, b2, w2r, bat2))
    seq = jnp.stack(seq_ts)

    wih = W_ih.T
    whh = W_hh.T
    wir, wiz, win = wih[:, 0:NC], wih[:, NC:2 * NC], wih[:, 2 * NC:3 * NC]
    whr, whz, whn = whh[:, 0:NC], whh[:, NC:2 * NC], whh[:, 2 * NC:3 * NC]
    br = (b_ih[0:NC] + b_hh[0:NC]).reshape(1, NC)
    bz = (b_ih[NC:2 * NC] + b_hh[NC:2 * NC]).reshape(1, NC)
    bin_ = b_ih[2 * NC:3 * NC].reshape(1, NC)
    bhn = b_hh[2 * NC:3 * NC].reshape(1, NC)

    out = pl.pallas_call(
        _gru_body,
        out_shape=jax.ShapeDtypeStruct((T, B, NC), jnp.float32),
    )(seq, wir, wiz, win, whr, whz, whn, br, bz, bin_, bhn)
    return out


# speedup vs baseline: 1.0813x; 1.0360x over previous
import functools

import jax
import jax.numpy as jnp
from jax import lax
from jax.experimental import pallas as pl
from jax.experimental.pallas import tpu as pltpu
from jax.experimental.pallas import tpu_sc as plsc

T = 4
N = 10000
E = 320000
B = 16
F = 128
NC = 16

NCORE = 2
NSUB = 16
NTILE = NCORE * NSUB
EPT = E // NTILE
K = 80
NCHUNK = EPT // K
RCH = 400
NRCH = N // RCH


def _agg_body(with_cnt, *refs):
    if with_cnt:
        (table, esrc, edst, zf, zc, onesh, out, cnt_out,
         acc, cacc, srca, dstv0, dstv1, gbuf0, gbuf1, onesb,
         gsem0, gsem1, ssem0, ssem1, csem0, csem1, isem0, isem1) = refs
    else:
        (table, esrc, edst, zf, out,
         acc, srca, dstv0, dstv1, gbuf0, gbuf1,
         gsem0, gsem1, ssem0, ssem1, isem0, isem1) = refs
    c = lax.axis_index("c")
    s = lax.axis_index("s")
    wid = c * NSUB + s
    nfull = NRCH // NSUB
    nextra = NRCH - nfull * NSUB

    if with_cnt:
        pltpu.sync_copy(onesh, onesb)

    def g_desc(ch, buf, sem):
        off = pl.multiple_of(ch * K, 8)
        return pltpu.make_async_copy(
            table.at[srca.at[pl.ds(off, K)]], buf, sem)

    def i_desc(ch, buf, sem):
        return pltpu.make_async_copy(edst.at[wid, ch], buf, sem)

    pltpu.sync_copy(esrc.at[pl.ds(wid * EPT, EPT)], srca)
    pltpu.sync_copy(edst.at[wid, 0], dstv0)
    g_desc(0, gbuf0, gsem0).start()
    i_desc(1, dstv1, isem1).start()

    def _zero(rr):
        pltpu.sync_copy(zf, acc.at[pl.ds(rr, RCH)])
        if with_cnt:
            pltpu.sync_copy(zc, cacc.at[pl.ds(rr, RCH)])

    for j in range(nfull):
        _zero(pl.multiple_of((s + j * NSUB) * RCH, 8))

    @pl.when(s < nextra)
    def _zx():
        _zero(pl.multiple_of((s + nfull * NSUB) * RCH, 8))
    plsc.subcore_barrier()


    def body2(it, carry):
        g = it * 2
        g_desc(g + 1, gbuf1, gsem1).start()
        g_desc(g, gbuf0, gsem0).wait()
        d0 = pltpu.async_copy(gbuf0, acc.at[dstv0], ssem0, add=True)
        if with_cnt:
            c0 = pltpu.async_copy(onesb, cacc.at[dstv0], csem0, add=True)
        i_desc(g + 1, dstv1, isem1).wait()
        d0.wait()
        if with_cnt:
            c0.wait()

        @pl.when(g + 2 < NCHUNK)
        def _n0():
            i_desc(g + 2, dstv0, isem0).start()
            g_desc(g + 2, gbuf0, gsem0).start()
        g_desc(g + 1, gbuf1, gsem1).wait()
        d1 = pltpu.async_copy(gbuf1, acc.at[dstv1], ssem1, add=True)
        if with_cnt:
            c1 = pltpu.async_copy(onesb, cacc.at[dstv1], csem1, add=True)

        @pl.when(g + 2 < NCHUNK)
        def _w0():
            i_desc(g + 2, dstv0, isem0).wait()
        d1.wait()
        if with_cnt:
            c1.wait()

        @pl.when(g + 3 < NCHUNK)
        def _n1():
            i_desc(g + 3, dstv1, isem1).start()
        return carry

    lax.fori_loop(0, NCHUNK // 2, body2, 0)
    g_desc(NCHUNK - 1, gbuf0, gsem0).wait()
    dl = pltpu.async_copy(gbuf0, acc.at[dstv0], ssem0, add=True)
    if with_cnt:
        cl = pltpu.async_copy(onesb, cacc.at[dstv0], csem0, add=True)
    dl.wait()
    if with_cnt:
        cl.wait()
    plsc.subcore_barrier()

    def _wout(rr):
        pltpu.sync_copy(acc.at[pl.ds(rr, RCH)], out.at[c, pl.ds(rr, RCH)])
        if with_cnt:
            pltpu.sync_copy(cacc.at[pl.ds(rr, RCH)],
                            cnt_out.at[c, pl.ds(rr, RCH)])

    for j in range(nfull):
        _wout(pl.multiple_of((s + j * NSUB) * RCH, 8))

    @pl.when(s < nextra)
    def _wx():
        _wout(pl.multiple_of((s + nfull * NSUB) * RCH, 8))


def _make_agg(with_cnt):
    mesh = plsc.VectorSubcoreMesh(core_axis_name="c", subcore_axis_name="s")
    out_type = [jax.ShapeDtypeStruct((NCORE, N, F), jnp.float32)]
    scratch = [
        pltpu.VMEM_SHARED((N, F), jnp.float32),
    ]
    if with_cnt:
        out_type.append(jax.ShapeDtypeStruct((NCORE, N, 16), jnp.float32))
        scratch.append(pltpu.VMEM_SHARED((N, 16), jnp.float32))
    scratch += [
        pltpu.VMEM((EPT,), jnp.int32),
        pltpu.VMEM((K,), jnp.int32),
        pltpu.VMEM((K,), jnp.int32),
        pltpu.VMEM((K, F), jnp.float32),
        pltpu.VMEM((K, F), jnp.float32),
    ]
    if with_cnt:
        scratch.append(pltpu.VMEM((K, 16), jnp.float32))
    nsem = 8 if with_cnt else 6
    scratch += [pltpu.SemaphoreType.DMA] * nsem
    return pl.kernel(
        functools.partial(_agg_body, with_cnt),
        out_type=tuple(out_type) if with_cnt else out_type[0],
        mesh=mesh,
        scratch_types=scratch,
        compiler_params=pltpu.CompilerParams(use_tc_tiling_on_sc=False),
    )


BLKR = 400
NBLK = N // BLKR


def _sage1_body(s_ref, c_ref, x_ref, wl_ref, b_ref, wr_ref, o_ref):
    ssum = s_ref[0] + s_ref[1]
    cnt = c_ref[0, :, 0:1] + c_ref[1, :, 0:1]
    m = ssum * (1.0 / jnp.maximum(cnt, 1.0))
    h = (jnp.dot(m, wl_ref[...], preferred_element_type=jnp.float32)
         + b_ref[...]
         + jnp.dot(x_ref[...], wr_ref[...], preferred_element_type=jnp.float32))
    o_ref[...] = jnp.maximum(h, 0.0)


def _sage2_body(s_ref, c_ref, x_ref, wl_ref, b_ref, wr_ref, bat_ref, o_ref):
    r = pl.program_id(0)

    @pl.when(r == 0)
    def _init():
        o_ref[...] = jnp.full((B, F), -jnp.inf, jnp.float32)

    ssum = s_ref[0] + s_ref[1]
    cnt = c_ref[0, :, 0:1] + c_ref[1, :, 0:1]
    m = ssum * (1.0 / jnp.maximum(cnt, 1.0))
    h = (jnp.dot(m, wl_ref[...], preferred_element_type=jnp.float32)
         + b_ref[...]
         + jnp.dot(x_ref[...], wr_ref[...], preferred_element_type=jnp.float32))
    bat = bat_ref[...]
    bmin = jnp.min(bat)
    bmax = jnp.max(bat)
    for b in range(B):
        @pl.when((b >= bmin) & (b <= bmax))
        def _upd(b=b):
            vals = jnp.where(bat == b, h, -jnp.inf)
            mb = jnp.max(vals, axis=0, keepdims=True)
            o_ref[b:b + 1, :] = jnp.maximum(o_ref[b:b + 1, :], mb)


def _gru_body(seq_ref, wir, wiz, win, whr, whz, whn, br, bz, bin_, bhn,
              o_ref):
    f32 = jnp.float32
    h = jnp.zeros((B, NC), f32)
    for t in range(T):
        xt = seq_ref[t]
        ir = jnp.dot(xt, wir[...], preferred_element_type=f32)
        iz = jnp.dot(xt, wiz[...], preferred_element_type=f32)
        in_ = jnp.dot(xt, win[...], preferred_element_type=f32)
        r = jax.nn.sigmoid(ir + jnp.dot(h, whr[...], preferred_element_type=f32) + br[...])
        z = jax.nn.sigmoid(iz + jnp.dot(h, whz[...], preferred_element_type=f32) + bz[...])
        n = jnp.tanh(in_ + bin_[...] + r * (jnp.dot(h, whn[...], preferred_element_type=f32) + bhn[...]))
        h = (1.0 - z) * n + z * h
        mx = jnp.max(h, axis=1, keepdims=True)
        lse = mx + jnp.log(jnp.sum(jnp.exp(h - mx), axis=1, keepdims=True))
        o_ref[t] = h - lse


def kernel(x, edge_index, batch, W1_l, b1_l, W1_r, W2_l, b2_l, W2_r,
           W_ih, W_hh, b_ih, b_hh):
    zf = jnp.zeros((RCH, F), jnp.float32)
    zc = jnp.zeros((RCH, 16), jnp.float32)
    onesh = jnp.ones((K, 16), jnp.float32)

    esrc = edge_index[:, 0, :]
    edst = edge_index[:, 1, :].reshape(T, NTILE, NCHUNK, K)

    agg1 = _make_agg(True)
    agg2 = _make_agg(False)

    w1l = W1_l.T
    w1r = W1_r.T
    b1 = b1_l.reshape(1, F)
    w2l = W2_l.T
    w2r = W2_r.T
    b2 = b2_l.reshape(1, F)
    bat2 = batch.reshape(N, 1)

    sage1 = pl.pallas_call(
        _sage1_body,
        grid=(NBLK,),
        in_specs=[
            pl.BlockSpec((NCORE, BLKR, F), lambda r: (0, r, 0)),
            pl.BlockSpec((NCORE, BLKR, 16), lambda r: (0, r, 0)),
            pl.BlockSpec((BLKR, F), lambda r: (r, 0)),
            pl.BlockSpec((F, F), lambda r: (0, 0)),
            pl.BlockSpec((1, F), lambda r: (0, 0)),
            pl.BlockSpec((F, F), lambda r: (0, 0)),
        ],
        out_specs=pl.BlockSpec((BLKR, F), lambda r: (r, 0)),
        out_shape=jax.ShapeDtypeStruct((N, F), jnp.float32),
    )
    sage2 = pl.pallas_call(
        _sage2_body,
        grid=(NBLK,),
        in_specs=[
            pl.BlockSpec((NCORE, BLKR, F), lambda r: (0, r, 0)),
            pl.BlockSpec((NCORE, BLKR, 16), lambda r: (0, r, 0)),
            pl.BlockSpec((BLKR, F), lambda r: (r, 0)),
            pl.BlockSpec((F, F), lambda r: (0, 0)),
            pl.BlockSpec((1, F), lambda r: (0, 0)),
            pl.BlockSpec((F, F), lambda r: (0, 0)),
            pl.BlockSpec((BLKR, 1), lambda r: (r, 0)),
        ],
        out_specs=pl.BlockSpec((B, F), lambda r: (0, 0)),
        out_shape=jax.ShapeDtypeStruct((B, F), jnp.float32),
    )

    seq_ts = []
    for t in range(T):
        s1, cnt = agg1(x[t], esrc[t], edst[t], zf, zc, onesh)
        h1 = sage1(s1, cnt, x[t], w1l, b1, w1r)
        s2 = agg2(h1, esrc[t], edst[t], zf)
        seq_ts.append(sage2(s2, cnt, h1, w2l, b2, w2r, bat2))
    seq = jnp.stack(seq_ts)

    wih = W_ih.T
    whh = W_hh.T
    wir, wiz, win = wih[:, 0:NC], wih[:, NC:2 * NC], wih[:, 2 * NC:3 * NC]
    whr, whz, whn = whh[:, 0:NC], whh[:, NC:2 * NC], whh[:, 2 * NC:3 * NC]
    br = (b_ih[0:NC] + b_hh[0:NC]).reshape(1, NC)
    bz = (b_ih[NC:2 * NC] + b_hh[NC:2 * NC]).reshape(1, NC)
    bin_ = b_ih[2 * NC:3 * NC].reshape(1, NC)
    bhn = b_hh[2 * NC:3 * NC].reshape(1, NC)

    out = pl.pallas_call(
        _gru_body,
        out_shape=jax.ShapeDtypeStruct((T, B, NC), jnp.float32),
    )(seq, wir, wiz, win, whr, whz, whn, br, bz, bin_, bhn)
    return out


# speedup vs baseline: 1.0926x; 1.0104x over previous
import functools

import jax
import jax.numpy as jnp
from jax import lax
from jax.experimental import pallas as pl
from jax.experimental.pallas import tpu as pltpu
from jax.experimental.pallas import tpu_sc as plsc

T = 4
N = 10000
E = 320000
B = 16
F = 128
NC = 16

NCORE = 2
NSUB = 16
NTILE = NCORE * NSUB
EPT = E // NTILE
K = 80
NCHUNK = EPT // K
RCH = 2000
NRCH = N // RCH


def _agg_body(with_cnt, *refs):
    if with_cnt:
        (table, esrc, edst, zf, zc, onesh, out, cnt_out,
         acc, cacc, srca, dstv0, dstv1, gbuf0, gbuf1, onesb,
         gsem0, gsem1, ssem0, ssem1, csem0, csem1, isem0, isem1) = refs
    else:
        (table, esrc, edst, zf, out,
         acc, srca, dstv0, dstv1, gbuf0, gbuf1,
         gsem0, gsem1, ssem0, ssem1, isem0, isem1) = refs
    c = lax.axis_index("c")
    s = lax.axis_index("s")
    wid = c * NSUB + s
    nfull = NRCH // NSUB
    nextra = NRCH - nfull * NSUB

    if with_cnt:
        pltpu.sync_copy(onesh, onesb)

    def g_desc(ch, buf, sem):
        off = pl.multiple_of(ch * K, 8)
        return pltpu.make_async_copy(
            table.at[srca.at[pl.ds(off, K)]], buf, sem)

    def i_desc(ch, buf, sem):
        return pltpu.make_async_copy(edst.at[wid, ch], buf, sem)

    pltpu.sync_copy(esrc.at[pl.ds(wid * EPT, EPT)], srca)
    pltpu.sync_copy(edst.at[wid, 0], dstv0)
    g_desc(0, gbuf0, gsem0).start()
    i_desc(1, dstv1, isem1).start()

    def _zero(rr):
        pltpu.sync_copy(zf, acc.at[pl.ds(rr, RCH)])
        if with_cnt:
            pltpu.sync_copy(zc, cacc.at[pl.ds(rr, RCH)])

    for j in range(nfull):
        _zero(pl.multiple_of((s + j * NSUB) * RCH, 8))

    @pl.when(s < nextra)
    def _zx():
        _zero(pl.multiple_of((s + nfull * NSUB) * RCH, 8))
    plsc.subcore_barrier()


    def body2(it, carry):
        g = it * 2
        g_desc(g + 1, gbuf1, gsem1).start()
        g_desc(g, gbuf0, gsem0).wait()
        d0 = pltpu.async_copy(gbuf0, acc.at[dstv0], ssem0, add=True)
        if with_cnt:
            c0 = pltpu.async_copy(onesb, cacc.at[dstv0], csem0, add=True)
        i_desc(g + 1, dstv1, isem1).wait()
        d0.wait()
        if with_cnt:
            c0.wait()

        @pl.when(g + 2 < NCHUNK)
        def _n0():
            i_desc(g + 2, dstv0, isem0).start()
            g_desc(g + 2, gbuf0, gsem0).start()
        g_desc(g + 1, gbuf1, gsem1).wait()
        d1 = pltpu.async_copy(gbuf1, acc.at[dstv1], ssem1, add=True)
        if with_cnt:
            c1 = pltpu.async_copy(onesb, cacc.at[dstv1], csem1, add=True)

        @pl.when(g + 2 < NCHUNK)
        def _w0():
            i_desc(g + 2, dstv0, isem0).wait()
        d1.wait()
        if with_cnt:
            c1.wait()

        @pl.when(g + 3 < NCHUNK)
        def _n1():
            i_desc(g + 3, dstv1, isem1).start()
        return carry

    lax.fori_loop(0, NCHUNK // 2, body2, 0)
    g_desc(NCHUNK - 1, gbuf0, gsem0).wait()
    dl = pltpu.async_copy(gbuf0, acc.at[dstv0], ssem0, add=True)
    if with_cnt:
        cl = pltpu.async_copy(onesb, cacc.at[dstv0], csem0, add=True)
    dl.wait()
    if with_cnt:
        cl.wait()
    plsc.subcore_barrier()

    def _wout(rr):
        pltpu.sync_copy(acc.at[pl.ds(rr, RCH)], out.at[c, pl.ds(rr, RCH)])
        if with_cnt:
            pltpu.sync_copy(cacc.at[pl.ds(rr, RCH)],
                            cnt_out.at[c, pl.ds(rr, RCH)])

    for j in range(nfull):
        _wout(pl.multiple_of((s + j * NSUB) * RCH, 8))

    @pl.when(s < nextra)
    def _wx():
        _wout(pl.multiple_of((s + nfull * NSUB) * RCH, 8))


def _make_agg(with_cnt):
    mesh = plsc.VectorSubcoreMesh(core_axis_name="c", subcore_axis_name="s")
    out_type = [jax.ShapeDtypeStruct((NCORE, N, F), jnp.float32)]
    scratch = [
        pltpu.VMEM_SHARED((N, F), jnp.float32),
    ]
    if with_cnt:
        out_type.append(jax.ShapeDtypeStruct((NCORE, N, 16), jnp.float32))
        scratch.append(pltpu.VMEM_SHARED((N, 16), jnp.float32))
    scratch += [
        pltpu.VMEM((EPT,), jnp.int32),
        pltpu.VMEM((K,), jnp.int32),
        pltpu.VMEM((K,), jnp.int32),
        pltpu.VMEM((K, F), jnp.float32),
        pltpu.VMEM((K, F), jnp.float32),
    ]
    if with_cnt:
        scratch.append(pltpu.VMEM((K, 16), jnp.float32))
    nsem = 8 if with_cnt else 6
    scratch += [pltpu.SemaphoreType.DMA] * nsem
    return pl.kernel(
        functools.partial(_agg_body, with_cnt),
        out_type=tuple(out_type) if with_cnt else out_type[0],
        mesh=mesh,
        scratch_types=scratch,
        compiler_params=pltpu.CompilerParams(use_tc_tiling_on_sc=False),
    )


BLKR = 400
NBLK = N // BLKR


def _sage1_body(s_ref, c_ref, x_ref, wl_ref, b_ref, wr_ref, o_ref):
    ssum = s_ref[0] + s_ref[1]
    cnt = c_ref[0, :, 0:1] + c_ref[1, :, 0:1]
    m = ssum * (1.0 / jnp.maximum(cnt, 1.0))
    h = (jnp.dot(m, wl_ref[...], preferred_element_type=jnp.float32)
         + b_ref[...]
         + jnp.dot(x_ref[...], wr_ref[...], preferred_element_type=jnp.float32))
    o_ref[...] = jnp.maximum(h, 0.0)


def _sage2_body(s_ref, c_ref, x_ref, wl_ref, b_ref, wr_ref, bat_ref, o_ref):
    r = pl.program_id(0)

    @pl.when(r == 0)
    def _init():
        o_ref[...] = jnp.full((B, F), -jnp.inf, jnp.float32)

    ssum = s_ref[0] + s_ref[1]
    cnt = c_ref[0, :, 0:1] + c_ref[1, :, 0:1]
    m = ssum * (1.0 / jnp.maximum(cnt, 1.0))
    h = (jnp.dot(m, wl_ref[...], preferred_element_type=jnp.float32)
         + b_ref[...]
         + jnp.dot(x_ref[...], wr_ref[...], preferred_element_type=jnp.float32))
    bat = bat_ref[...]
    bmin = jnp.min(bat)
    bmax = jnp.max(bat)
    for b in range(B):
        @pl.when((b >= bmin) & (b <= bmax))
        def _upd(b=b):
            vals = jnp.where(bat == b, h, -jnp.inf)
            mb = jnp.max(vals, axis=0, keepdims=True)
            o_ref[b:b + 1, :] = jnp.maximum(o_ref[b:b + 1, :], mb)


def _gru_body(seq_ref, wir, wiz, win, whr, whz, whn, br, bz, bin_, bhn,
              o_ref):
    f32 = jnp.float32
    h = jnp.zeros((B, NC), f32)
    for t in range(T):
        xt = seq_ref[t]
        ir = jnp.dot(xt, wir[...], preferred_element_type=f32)
        iz = jnp.dot(xt, wiz[...], preferred_element_type=f32)
        in_ = jnp.dot(xt, win[...], preferred_element_type=f32)
        r = jax.nn.sigmoid(ir + jnp.dot(h, whr[...], preferred_element_type=f32) + br[...])
        z = jax.nn.sigmoid(iz + jnp.dot(h, whz[...], preferred_element_type=f32) + bz[...])
        n = jnp.tanh(in_ + bin_[...] + r * (jnp.dot(h, whn[...], preferred_element_type=f32) + bhn[...]))
        h = (1.0 - z) * n + z * h
        mx = jnp.max(h, axis=1, keepdims=True)
        lse = mx + jnp.log(jnp.sum(jnp.exp(h - mx), axis=1, keepdims=True))
        o_ref[t] = h - lse


def kernel(x, edge_index, batch, W1_l, b1_l, W1_r, W2_l, b2_l, W2_r,
           W_ih, W_hh, b_ih, b_hh):
    zf = jnp.zeros((RCH, F), jnp.float32)
    zc = jnp.zeros((RCH, 16), jnp.float32)
    onesh = jnp.ones((K, 16), jnp.float32)

    esrc = edge_index[:, 0, :]
    edst = edge_index[:, 1, :].reshape(T, NTILE, NCHUNK, K)

    agg1 = _make_agg(True)
    agg2 = _make_agg(False)

    w1l = W1_l.T
    w1r = W1_r.T
    b1 = b1_l.reshape(1, F)
    w2l = W2_l.T
    w2r = W2_r.T
    b2 = b2_l.reshape(1, F)
    bat2 = batch.reshape(N, 1)

    sage1 = pl.pallas_call(
        _sage1_body,
        grid=(NBLK,),
        in_specs=[
            pl.BlockSpec((NCORE, BLKR, F), lambda r: (0, r, 0)),
            pl.BlockSpec((NCORE, BLKR, 16), lambda r: (0, r, 0)),
            pl.BlockSpec((BLKR, F), lambda r: (r, 0)),
            pl.BlockSpec((F, F), lambda r: (0, 0)),
            pl.BlockSpec((1, F), lambda r: (0, 0)),
            pl.BlockSpec((F, F), lambda r: (0, 0)),
        ],
        out_specs=pl.BlockSpec((BLKR, F), lambda r: (r, 0)),
        out_shape=jax.ShapeDtypeStruct((N, F), jnp.float32),
    )
    sage2 = pl.pallas_call(
        _sage2_body,
        grid=(NBLK,),
        in_specs=[
            pl.BlockSpec((NCORE, BLKR, F), lambda r: (0, r, 0)),
            pl.BlockSpec((NCORE, BLKR, 16), lambda r: (0, r, 0)),
            pl.BlockSpec((BLKR, F), lambda r: (r, 0)),
            pl.BlockSpec((F, F), lambda r: (0, 0)),
            pl.BlockSpec((1, F), lambda r: (0, 0)),
            pl.BlockSpec((F, F), lambda r: (0, 0)),
            pl.BlockSpec((BLKR, 1), lambda r: (r, 0)),
        ],
        out_specs=pl.BlockSpec((B, F), lambda r: (0, 0)),
        out_shape=jax.ShapeDtypeStruct((B, F), jnp.float32),
    )

    seq_ts = []
    for t in range(T):
        s1, cnt = agg1(x[t], esrc[t], edst[t], zf, zc, onesh)
        h1 = sage1(s1, cnt, x[t], w1l, b1, w1r)
        s2 = agg2(h1, esrc[t], edst[t], zf)
        seq_ts.append(sage2(s2, cnt, h1, w2l, b2, w2r, bat2))
    seq = jnp.stack(seq_ts)

    wih = W_ih.T
    whh = W_hh.T
    wir, wiz, win = wih[:, 0:NC], wih[:, NC:2 * NC], wih[:, 2 * NC:3 * NC]
    whr, whz, whn = whh[:, 0:NC], whh[:, NC:2 * NC], whh[:, 2 * NC:3 * NC]
    br = (b_ih[0:NC] + b_hh[0:NC]).reshape(1, NC)
    bz = (b_ih[NC:2 * NC] + b_hh[NC:2 * NC]).reshape(1, NC)
    bin_ = b_ih[2 * NC:3 * NC].reshape(1, NC)
    bhn = b_hh[2 * NC:3 * NC].reshape(1, NC)

    out = pl.pallas_call(
        _gru_body,
        out_shape=jax.ShapeDtypeStruct((T, B, NC), jnp.float32),
    )(seq, wir, wiz, win, whr, whz, whn, br, bz, bin_, bhn)
    return out
